# baseline (speedup 1.0000x reference)
import functools

import jax
import jax.numpy as jnp
from jax import lax
from jax.experimental import pallas as pl
from jax.experimental.pallas import tpu as pltpu
from jax.experimental.pallas import tpu_sc as plsc

_DIM = 64
_NC = 2
_NS = 16
_NW = _NC * _NS
_LANE = 128
_NGB = 3
_NTB = 2
_STEP = 6


def _make_gather(n_fields: int, n_btiles: int):
    n_units = n_fields * n_btiles
    assert n_units % _NW == 0
    upw = n_units // _NW
    n_outer = (upw + _STEP - 1) // _STEP
    mesh = plsc.VectorSubcoreMesh(core_axis_name="c", subcore_axis_name="s")

    @functools.partial(
        pl.kernel,
        out_type=jax.ShapeDtypeStruct(
            (n_fields, _DIM // 8, n_btiles, 8, _LANE), jnp.float32
        ),
        mesh=mesh,
        scratch_types=(
            [pltpu.VMEM((upw, _LANE), jnp.int32)]
            + [pltpu.VMEM((_LANE, 2 * _DIM), jnp.float32) for _ in range(_NGB)]
            + [pltpu.VMEM((_DIM // 8, 8, _LANE), jnp.float32) for _ in range(_NTB)]
            + [pltpu.SemaphoreType.DMA for _ in range(_NGB + _NTB)]
        ),
        compiler_params=pltpu.CompilerParams(
            use_tc_tiling_on_sc=False, needs_layout_passes=False
        ),
    )
    def gather_kernel(table_hbm, idx_hbm, out_hbm, idx_v, *scratch):
        gbufs = scratch[:_NGB]
        tbufs = scratch[_NGB:_NGB + _NTB]
        gsems = scratch[_NGB + _NTB:2 * _NGB + _NTB]
        wsems = scratch[2 * _NGB + _NTB:]
        wid = lax.axis_index("s") * _NC + lax.axis_index("c")
        pltpu.sync_copy(idx_hbm.at[wid], idx_v)

        iota = lax.iota(jnp.int32, 16)
        rows = [iota + 16 * k2 for k2 in range(8)]

        def start_gather(k, g):
            pltpu.async_copy(table_hbm.at[idx_v.at[k]], gbufs[g], gsems[g])

        for g in range(_NGB - 1):
            start_gather(g, g)

        def unit_block(t6, carry):
            for k6 in range(_STEP):
                g = k6 % _NGB
                t = k6 % _NTB
                k = t6 * _STEP + k6
                u = wid * upw + k
                f = u // n_btiles
                bc = u % n_btiles

                @pl.when(k < upw)
                def _():
                    kn = k + _NGB - 1

                    @pl.when(kn < upw)
                    def _():
                        start_gather(kn, (k6 + _NGB - 1) % _NGB)

                    pltpu.make_async_copy(
                        table_hbm.at[idx_v.at[k]], gbufs[g], gsems[g]
                    ).wait()

                    @pl.when(k >= _NTB)
                    def _():
                        pltpu.make_async_copy(
                            tbufs[t], out_hbm.at[f, :, bc], wsems[t]
                        ).wait()

                    @plsc.parallel_loop(0, _DIM, 1, unroll=8)
                    def col(d0):
                        dvec = (d0 + iota) & (_DIM - 1)
                        tr = dvec >> 3
                        s = dvec & 7
                        for k2 in range(8):
                            vals = plsc.load_gather(gbufs[g], [rows[k2], dvec])
                            plsc.store_scatter(
                                tbufs[t], [tr, s, rows[k2]], vals
                            )
                    pltpu.async_copy(tbufs[t], out_hbm.at[f, :, bc], wsems[t])

            return carry

        lax.fori_loop(0, n_outer, unit_block, 0, unroll=False)

        for i in range(_NTB):
            u = wid * upw + (upw - _NTB + i)
            pltpu.make_async_copy(
                tbufs[(upw - _NTB + i) % _NTB],
                out_hbm.at[u // n_btiles, :, u % n_btiles],
                wsems[(upw - _NTB + i) % _NTB],
            ).wait()

    return gather_kernel


def kernel(metadata_ids, emb_weight):
    batch, n_fields = metadata_ids.shape
    n_btiles = batch // _LANE
    ids_t = metadata_ids.T.astype(jnp.int32)
    idx = ids_t.reshape(_NW, (n_fields * n_btiles) // _NW, _LANE)
    table_pad = jnp.pad(emb_weight, ((0, 0), (0, 2 * _DIM - emb_weight.shape[1])))
    out5 = _make_gather(n_fields, n_btiles)(table_pad, idx)
    return out5.transpose(2, 4, 0, 1, 3).reshape(batch, n_fields, _DIM)


# speedup vs baseline: 1.0927x; 1.0927x over previous
import functools

import jax
import jax.numpy as jnp
from jax import lax
from jax.experimental import pallas as pl
from jax.experimental.pallas import tpu as pltpu
from jax.experimental.pallas import tpu_sc as plsc

_DIM = 64
_NC = 2
_NS = 16
_NW = _NC * _NS
_LANE = 128
_NGB = 3
_NTB = 2
_STEP = 6


def _make_gather(n_fields: int, n_btiles: int):
    n_units = n_fields * n_btiles
    assert n_units % _NW == 0
    upw = n_units // _NW
    n_outer = (upw + _STEP - 1) // _STEP
    mesh = plsc.VectorSubcoreMesh(core_axis_name="c", subcore_axis_name="s")

    @functools.partial(
        pl.kernel,
        out_type=jax.ShapeDtypeStruct(
            (n_fields, _DIM // 8, n_btiles, 8, _LANE), jnp.float32
        ),
        mesh=mesh,
        scratch_types=(
            [pltpu.VMEM((upw, _LANE), jnp.int32)]
            + [pltpu.VMEM((_LANE, _DIM), jnp.float32) for _ in range(_NGB)]
            + [pltpu.VMEM((_DIM // 8, 8, _LANE), jnp.float32) for _ in range(_NTB)]
            + [pltpu.SemaphoreType.DMA for _ in range(_NGB + _NTB)]
        ),
        compiler_params=pltpu.CompilerParams(
            use_tc_tiling_on_sc=False, needs_layout_passes=False
        ),
    )
    def gather_kernel(table_hbm, idx_hbm, out_hbm, idx_v, *scratch):
        gbufs = scratch[:_NGB]
        tbufs = scratch[_NGB:_NGB + _NTB]
        gsems = scratch[_NGB + _NTB:2 * _NGB + _NTB]
        wsems = scratch[2 * _NGB + _NTB:]
        wid = lax.axis_index("s") * _NC + lax.axis_index("c")
        pltpu.sync_copy(idx_hbm.at[wid], idx_v)

        iota = lax.iota(jnp.int32, 16)
        rows = [iota + 16 * k2 for k2 in range(8)]

        def start_gather(k, g):
            pltpu.async_copy(table_hbm.at[idx_v.at[k]], gbufs[g], gsems[g])

        for g in range(_NGB - 1):
            start_gather(g, g)

        def unit_block(t6, carry):
            for k6 in range(_STEP):
                g = k6 % _NGB
                t = k6 % _NTB
                k = t6 * _STEP + k6
                u = wid * upw + k
                f = u // n_btiles
                bc = u % n_btiles

                @pl.when(k < upw)
                def _():
                    kn = k + _NGB - 1

                    @pl.when(kn < upw)
                    def _():
                        start_gather(kn, (k6 + _NGB - 1) % _NGB)

                    pltpu.make_async_copy(
                        table_hbm.at[idx_v.at[k]], gbufs[g], gsems[g]
                    ).wait()

                    @pl.when(k >= _NTB)
                    def _():
                        pltpu.make_async_copy(
                            tbufs[t], out_hbm.at[f, :, bc], wsems[t]
                        ).wait()

                    @plsc.parallel_loop(0, _DIM, 1, unroll=8)
                    def col(d0):
                        dvec = (d0 + iota) & (_DIM - 1)
                        tr = dvec >> 3
                        s = dvec & 7
                        for k2 in range(8):
                            vals = plsc.load_gather(gbufs[g], [rows[k2], dvec])
                            plsc.store_scatter(
                                tbufs[t], [tr, s, rows[k2]], vals
                            )
                    pltpu.async_copy(tbufs[t], out_hbm.at[f, :, bc], wsems[t])

            return carry

        lax.fori_loop(0, n_outer, unit_block, 0, unroll=False)

        for i in range(_NTB):
            u = wid * upw + (upw - _NTB + i)
            pltpu.make_async_copy(
                tbufs[(upw - _NTB + i) % _NTB],
                out_hbm.at[u // n_btiles, :, u % n_btiles],
                wsems[(upw - _NTB + i) % _NTB],
            ).wait()

    return gather_kernel


def kernel(metadata_ids, emb_weight):
    batch, n_fields = metadata_ids.shape
    n_btiles = batch // _LANE
    ids_t = metadata_ids.T.astype(jnp.int32)
    idx = (ids_t * 2).reshape(_NW, (n_fields * n_btiles) // _NW, _LANE)
    table_pad = jnp.pad(emb_weight, ((0, 0), (0, 2 * _DIM - emb_weight.shape[1])))
    table2 = table_pad.reshape(2 * emb_weight.shape[0], _DIM)
    out5 = _make_gather(n_fields, n_btiles)(table2, idx)
    return out5.transpose(2, 4, 0, 1, 3).reshape(batch, n_fields, _DIM)
